# initial kernel scaffold (unmeasured)
import jax
import jax.numpy as jnp
from jax import lax
from jax.experimental import pallas as pl
from jax.experimental.pallas import tpu as pltpu


def kernel(
    x,
):
    def body(*refs):
        pass

    out_shape = jax.ShapeDtypeStruct(..., jnp.float32)
    return pl.pallas_call(body, out_shape=out_shape)(...)


# baseline (device time: 13397 ns/iter reference)
import jax
import jax.numpy as jnp
from jax import lax
from jax.experimental import pallas as pl
from jax.experimental.pallas import tpu as pltpu


def kernel(x):
    m, n = x.shape

    def body(x_ref, out_ref, comm_ref, send_sems, recv_sems):
        my_x = lax.axis_index("x")
        my_y = lax.axis_index("y")
        y_nbr = (my_x, 1 - my_y)
        x_nbr = (1 - my_x, my_y)

        barrier_sem = pltpu.get_barrier_semaphore()
        for nbr in (y_nbr, x_nbr):
            pl.semaphore_signal(
                barrier_sem, inc=1,
                device_id=nbr, device_id_type=pl.DeviceIdType.MESH,
            )
        pl.semaphore_wait(barrier_sem, 2)

        step1 = pltpu.make_async_remote_copy(
            src_ref=x_ref,
            dst_ref=comm_ref.at[0],
            send_sem=send_sems.at[0],
            recv_sem=recv_sems.at[0],
            device_id=y_nbr,
            device_id_type=pl.DeviceIdType.MESH,
        )
        step1.start()
        step1.wait()
        out_ref[:, :] = x_ref[:, :] + comm_ref[0, :, :]

        step2 = pltpu.make_async_remote_copy(
            src_ref=out_ref,
            dst_ref=comm_ref.at[1],
            send_sem=send_sems.at[1],
            recv_sem=recv_sems.at[1],
            device_id=x_nbr,
            device_id_type=pl.DeviceIdType.MESH,
        )
        step2.start()
        step2.wait()
        out_ref[:, :] = out_ref[:, :] + comm_ref[1, :, :]

    return pl.pallas_call(
        body,
        out_shape=jax.ShapeDtypeStruct((m, n), x.dtype),
        in_specs=[pl.BlockSpec(memory_space=pltpu.VMEM)],
        out_specs=pl.BlockSpec(memory_space=pltpu.VMEM),
        scratch_shapes=[
            pltpu.VMEM((2, m, n), x.dtype),
            pltpu.SemaphoreType.DMA((2,)),
            pltpu.SemaphoreType.DMA((2,)),
        ],
        compiler_params=pltpu.CompilerParams(collective_id=0),
    )(x)


# device time: 10641 ns/iter; 1.2590x vs baseline; 1.2590x over previous
import jax
import jax.numpy as jnp
from jax import lax
from jax.experimental import pallas as pl
from jax.experimental.pallas import tpu as pltpu


def kernel(x):
    m, n = x.shape
    half = m // 2

    def body(x_ref, out_ref, part_ref, comm_ref, send_sems, recv_sems):
        my_x = lax.axis_index("x")
        my_y = lax.axis_index("y")
        y_nbr = (my_x, 1 - my_y)
        x_nbr = (1 - my_x, my_y)

        barrier_sem = pltpu.get_barrier_semaphore()
        for nbr in (y_nbr, x_nbr):
            pl.semaphore_signal(
                barrier_sem, inc=1,
                device_id=nbr, device_id_type=pl.DeviceIdType.MESH,
            )
        pl.semaphore_wait(barrier_sem, 2)

        p1a = pltpu.make_async_remote_copy(
            src_ref=x_ref.at[pl.ds(0, half)],
            dst_ref=comm_ref.at[0],
            send_sem=send_sems.at[0],
            recv_sem=recv_sems.at[0],
            device_id=y_nbr,
            device_id_type=pl.DeviceIdType.MESH,
        )
        p1b = pltpu.make_async_remote_copy(
            src_ref=x_ref.at[pl.ds(half, half)],
            dst_ref=comm_ref.at[1],
            send_sem=send_sems.at[1],
            recv_sem=recv_sems.at[1],
            device_id=x_nbr,
            device_id_type=pl.DeviceIdType.MESH,
        )
        p1a.start()
        p1b.start()

        p1a.wait_recv()
        part_ref[pl.ds(0, half), :] = x_ref[pl.ds(0, half), :] + comm_ref[0, :, :]
        p2a = pltpu.make_async_remote_copy(
            src_ref=part_ref.at[pl.ds(0, half)],
            dst_ref=comm_ref.at[2],
            send_sem=send_sems.at[2],
            recv_sem=recv_sems.at[2],
            device_id=x_nbr,
            device_id_type=pl.DeviceIdType.MESH,
        )
        p2a.start()

        p1b.wait_recv()
        part_ref[pl.ds(half, half), :] = (
            x_ref[pl.ds(half, half), :] + comm_ref[1, :, :]
        )
        p2b = pltpu.make_async_remote_copy(
            src_ref=part_ref.at[pl.ds(half, half)],
            dst_ref=comm_ref.at[3],
            send_sem=send_sems.at[3],
            recv_sem=recv_sems.at[3],
            device_id=y_nbr,
            device_id_type=pl.DeviceIdType.MESH,
        )
        p2b.start()

        p2a.wait_recv()
        out_ref[pl.ds(0, half), :] = part_ref[pl.ds(0, half), :] + comm_ref[2, :, :]
        p2b.wait_recv()
        out_ref[pl.ds(half, half), :] = (
            part_ref[pl.ds(half, half), :] + comm_ref[3, :, :]
        )

        p1a.wait_send()
        p1b.wait_send()
        p2a.wait_send()
        p2b.wait_send()

    return pl.pallas_call(
        body,
        out_shape=jax.ShapeDtypeStruct((m, n), x.dtype),
        in_specs=[pl.BlockSpec(memory_space=pltpu.VMEM)],
        out_specs=pl.BlockSpec(memory_space=pltpu.VMEM),
        scratch_shapes=[
            pltpu.VMEM((m, n), x.dtype),
            pltpu.VMEM((4, half, n), x.dtype),
            pltpu.SemaphoreType.DMA((4,)),
            pltpu.SemaphoreType.DMA((4,)),
        ],
        compiler_params=pltpu.CompilerParams(collective_id=0),
    )(x)


# device time: 9972 ns/iter; 1.3435x vs baseline; 1.0671x over previous
import jax
import jax.numpy as jnp
from jax import lax
from jax.experimental import pallas as pl
from jax.experimental.pallas import tpu as pltpu

N_CHUNKS = 4


def kernel(x):
    m, n = x.shape
    ck = m // N_CHUNKS

    def body(x_ref, out_ref, part_ref, comm_ref, send_sems, recv_sems):
        my_x = lax.axis_index("x")
        my_y = lax.axis_index("y")
        y_nbr = (my_x, 1 - my_y)
        x_nbr = (1 - my_x, my_y)
        p1_dev = {0: y_nbr, 1: y_nbr, 2: x_nbr, 3: x_nbr}
        p2_dev = {0: x_nbr, 1: x_nbr, 2: y_nbr, 3: y_nbr}

        barrier_sem = pltpu.get_barrier_semaphore()
        for nbr in (y_nbr, x_nbr):
            pl.semaphore_signal(
                barrier_sem, inc=1,
                device_id=nbr, device_id_type=pl.DeviceIdType.MESH,
            )
        pl.semaphore_wait(barrier_sem, 2)

        p1 = []
        for c in range(N_CHUNKS):
            rdma = pltpu.make_async_remote_copy(
                src_ref=x_ref.at[pl.ds(c * ck, ck)],
                dst_ref=comm_ref.at[c],
                send_sem=send_sems.at[c],
                recv_sem=recv_sems.at[c],
                device_id=p1_dev[c],
                device_id_type=pl.DeviceIdType.MESH,
            )
            rdma.start()
            p1.append(rdma)

        p2 = [None] * N_CHUNKS
        for c in (0, 2, 1, 3):
            p1[c].wait_recv()
            part_ref[pl.ds(c * ck, ck), :] = (
                x_ref[pl.ds(c * ck, ck), :] + comm_ref[c, :, :]
            )
            rdma = pltpu.make_async_remote_copy(
                src_ref=part_ref.at[pl.ds(c * ck, ck)],
                dst_ref=comm_ref.at[N_CHUNKS + c],
                send_sem=send_sems.at[N_CHUNKS + c],
                recv_sem=recv_sems.at[N_CHUNKS + c],
                device_id=p2_dev[c],
                device_id_type=pl.DeviceIdType.MESH,
            )
            rdma.start()
            p2[c] = rdma

        for c in (0, 2, 1, 3):
            p2[c].wait_recv()
            out_ref[pl.ds(c * ck, ck), :] = (
                part_ref[pl.ds(c * ck, ck), :] + comm_ref[N_CHUNKS + c, :, :]
            )

        for rdma in p1 + p2:
            rdma.wait_send()

    return pl.pallas_call(
        body,
        out_shape=jax.ShapeDtypeStruct((m, n), x.dtype),
        in_specs=[pl.BlockSpec(memory_space=pltpu.VMEM)],
        out_specs=pl.BlockSpec(memory_space=pltpu.VMEM),
        scratch_shapes=[
            pltpu.VMEM((m, n), x.dtype),
            pltpu.VMEM((2 * N_CHUNKS, ck, n), x.dtype),
            pltpu.SemaphoreType.DMA((2 * N_CHUNKS,)),
            pltpu.SemaphoreType.DMA((2 * N_CHUNKS,)),
        ],
        compiler_params=pltpu.CompilerParams(collective_id=0),
    )(x)


# device time: 9773 ns/iter; 1.3708x vs baseline; 1.0204x over previous
import jax
import jax.numpy as jnp
from jax import lax
from jax.experimental import pallas as pl
from jax.experimental.pallas import tpu as pltpu

N_CHUNKS = 8


def kernel(x):
    m, n = x.shape
    ck = m // N_CHUNKS

    def body(x_ref, out_ref, part_ref, comm_ref, send_sems, recv_sems):
        my_x = lax.axis_index("x")
        my_y = lax.axis_index("y")
        y_nbr = (my_x, 1 - my_y)
        x_nbr = (1 - my_x, my_y)
        h = N_CHUNKS // 2
        p1_dev = {c: (y_nbr if c < h else x_nbr) for c in range(N_CHUNKS)}
        p2_dev = {c: (x_nbr if c < h else y_nbr) for c in range(N_CHUNKS)}
        arrival = [c for pair in zip(range(h), range(h, N_CHUNKS)) for c in pair]

        barrier_sem = pltpu.get_barrier_semaphore()
        for nbr in (y_nbr, x_nbr):
            pl.semaphore_signal(
                barrier_sem, inc=1,
                device_id=nbr, device_id_type=pl.DeviceIdType.MESH,
            )
        pl.semaphore_wait(barrier_sem, 2)

        p1 = []
        for c in range(N_CHUNKS):
            rdma = pltpu.make_async_remote_copy(
                src_ref=x_ref.at[pl.ds(c * ck, ck)],
                dst_ref=comm_ref.at[c],
                send_sem=send_sems.at[c],
                recv_sem=recv_sems.at[c],
                device_id=p1_dev[c],
                device_id_type=pl.DeviceIdType.MESH,
            )
            rdma.start()
            p1.append(rdma)

        p2 = [None] * N_CHUNKS
        for c in arrival:
            p1[c].wait_recv()
            part_ref[pl.ds(c * ck, ck), :] = (
                x_ref[pl.ds(c * ck, ck), :] + comm_ref[c, :, :]
            )
            rdma = pltpu.make_async_remote_copy(
                src_ref=part_ref.at[pl.ds(c * ck, ck)],
                dst_ref=comm_ref.at[N_CHUNKS + c],
                send_sem=send_sems.at[N_CHUNKS + c],
                recv_sem=recv_sems.at[N_CHUNKS + c],
                device_id=p2_dev[c],
                device_id_type=pl.DeviceIdType.MESH,
            )
            rdma.start()
            p2[c] = rdma

        for c in arrival:
            p2[c].wait_recv()
            out_ref[pl.ds(c * ck, ck), :] = (
                part_ref[pl.ds(c * ck, ck), :] + comm_ref[N_CHUNKS + c, :, :]
            )

        for rdma in p1 + p2:
            rdma.wait_send()

    return pl.pallas_call(
        body,
        out_shape=jax.ShapeDtypeStruct((m, n), x.dtype),
        in_specs=[pl.BlockSpec(memory_space=pltpu.VMEM)],
        out_specs=pl.BlockSpec(memory_space=pltpu.VMEM),
        scratch_shapes=[
            pltpu.VMEM((m, n), x.dtype),
            pltpu.VMEM((2 * N_CHUNKS, ck, n), x.dtype),
            pltpu.SemaphoreType.DMA((2 * N_CHUNKS,)),
            pltpu.SemaphoreType.DMA((2 * N_CHUNKS,)),
        ],
        compiler_params=pltpu.CompilerParams(collective_id=0),
    )(x)
